# baseline (speedup 1.0000x reference)
import jax
import jax.numpy as jnp
from jax.experimental import pallas as pl
from jax.experimental.pallas import tpu as pltpu

_BN_EPS = 4e-5


def _make_body(n_imgs, c_in, c_out, h_blk, w_dim, m_total, bn_eps):
    hw_blk = h_blk * w_dim

    def _body(x_ref, w_ref, g_ref, b_ref, o_ref,
              x_buf, gram_ref, s_ref, ws_ref, shift_ref):
        p = pl.program_id(0)
        i = pl.program_id(1)
        n = i // 2
        t = i % 2

        @pl.when(p == 0)
        def _stats():
            @pl.when(i == 0)
            def _init():
                gram_ref[...] = jnp.zeros_like(gram_ref)
                s_ref[...] = jnp.zeros_like(s_ref)

            xf = x_ref[0].reshape(c_in, hw_blk)
            xb = xf.astype(jnp.bfloat16)
            x_buf[n, t] = xb
            gram_ref[...] += jax.lax.dot_general(
                xb, xb, (((1,), (1,)), ((), ())),
                preferred_element_type=jnp.float32)
            s_ref[...] += jnp.sum(xf, axis=1, keepdims=True)

        @pl.when(p == 1)
        def _normalize():
            @pl.when(i == 0)
            def _fold():
                wf = w_ref[...]
                mean = jnp.dot(wf, s_ref[...],
                               preferred_element_type=jnp.float32) / m_total
                a = jnp.dot(wf, gram_ref[...],
                            preferred_element_type=jnp.float32)
                e2 = jnp.sum(a * wf, axis=1, keepdims=True) / m_total
                var = jnp.maximum(e2 - mean * mean, 0.0)
                inv_std = 1.0 / jnp.sqrt(var + bn_eps)
                scale = g_ref[...] * inv_std
                shift_ref[...] = b_ref[...] - mean * scale
                ws_ref[...] = (wf * scale).astype(jnp.bfloat16)

            y = jnp.dot(ws_ref[...], x_buf[n, t],
                        preferred_element_type=jnp.float32)
            z = jnp.maximum(y + shift_ref[...], 0.0).astype(o_ref.dtype)
            o_ref[0] = z.reshape(c_out, h_blk, w_dim)

    return _body


def kernel(x, conv_w, conv_b, gamma, beta):
    N, Cin, H, W = x.shape
    Cout = conv_w.shape[0]
    HW = H * W
    M = N * HW
    del conv_b

    w_mat = conv_w.reshape(Cout, Cin).astype(jnp.float32)
    g2 = gamma.astype(jnp.float32).reshape(Cout, 1)
    b2 = beta.astype(jnp.float32).reshape(Cout, 1)

    H_BLK = H // 2
    n_steps = 2 * N

    x_spec = pl.BlockSpec(
        (1, Cin, H_BLK, W),
        lambda p, i: (jnp.where(p == 0, i // 2, N - 1), 0,
                      jnp.where(p == 0, i % 2, 1), 0))
    o_spec = pl.BlockSpec(
        (1, Cout, H_BLK, W),
        lambda p, i: (jnp.where(p == 0, 0, i // 2), 0,
                      jnp.where(p == 0, 0, i % 2), 0))
    w_spec = pl.BlockSpec((Cout, Cin), lambda p, i: (0, 0))
    vec_spec = pl.BlockSpec((Cout, 1), lambda p, i: (0, 0))

    cost = pl.CostEstimate(
        flops=4 * M * Cin * Cout + 7 * M * Cout,
        transcendentals=Cout,
        bytes_accessed=M * Cin * 4 + M * Cout * 4 + Cout * Cin * 4)

    out = pl.pallas_call(
        _make_body(N, Cin, Cout, H_BLK, W, M, _BN_EPS),
        out_shape=jax.ShapeDtypeStruct((N, Cout, H, W), x.dtype),
        grid=(2, n_steps),
        in_specs=[x_spec, w_spec, vec_spec, vec_spec],
        out_specs=o_spec,
        scratch_shapes=[
            pltpu.VMEM((N, 2, Cin, H_BLK * W), jnp.bfloat16),
            pltpu.VMEM((Cin, Cin), jnp.float32),
            pltpu.VMEM((Cin, 1), jnp.float32),
            pltpu.VMEM((Cout, Cin), jnp.bfloat16),
            pltpu.VMEM((Cout, 1), jnp.float32),
        ],
        compiler_params=pltpu.CompilerParams(
            dimension_semantics=("arbitrary", "arbitrary"),
            vmem_limit_bytes=60 * 1024 * 1024),
        cost_estimate=cost,
    )(x, w_mat, g2, b2)

    return out


# speedup vs baseline: 1.9916x; 1.9916x over previous
import jax
import jax.numpy as jnp
from jax.experimental import pallas as pl
from jax.experimental.pallas import tpu as pltpu

_BN_EPS = 4e-5


def _make_body(n_imgs, m_total, bn_eps):
    def _body(x_ref, w_ref, g_ref, b_ref, o_ref,
              x_buf, gram_ref, s_ref, ws_ref, shift_ref):
        p = pl.program_id(0)
        n = pl.program_id(1)

        @pl.when(p == 0)
        def _stats():
            @pl.when(n == 0)
            def _init():
                gram_ref[...] = jnp.zeros_like(gram_ref)
                s_ref[...] = jnp.zeros_like(s_ref)

            xq = x_ref[0]
            x_buf[n] = xq
            gram_ref[...] += jax.lax.dot_general(
                xq, xq, (((1,), (1,)), ((), ())),
                preferred_element_type=jnp.float32)
            ones = jnp.ones((xq.shape[1], 128), jnp.bfloat16)
            s_ref[...] += jnp.dot(xq, ones,
                                  preferred_element_type=jnp.float32)

        @pl.when(p == 1)
        def _normalize():
            @pl.when(n == 0)
            def _fold():
                wf = w_ref[...]
                mean = jnp.dot(wf, s_ref[:, 0:1],
                               preferred_element_type=jnp.float32) / m_total
                a = jnp.dot(wf, gram_ref[...],
                            preferred_element_type=jnp.float32)
                e2 = jnp.sum(a * wf, axis=1, keepdims=True) / m_total
                var = jnp.maximum(e2 - mean * mean, 0.0)
                inv_std = 1.0 / jnp.sqrt(var + bn_eps)
                scale = g_ref[...] * inv_std
                shift_ref[...] = b_ref[...] - mean * scale
                ws_ref[...] = (wf * scale).astype(jnp.bfloat16)

            y = jnp.dot(ws_ref[...], x_buf[n],
                        preferred_element_type=jnp.float32)
            z = jnp.maximum(y + shift_ref[...], 0.0)
            o_ref[0] = z.astype(o_ref.dtype)

    return _body


def kernel(x, conv_w, conv_b, gamma, beta):
    N, Cin, H, W = x.shape
    Cout = conv_w.shape[0]
    HW = H * W
    M = N * HW
    del conv_b

    xb = x.reshape(N, Cin, HW).astype(jnp.bfloat16)
    w_mat = conv_w.reshape(Cout, Cin).astype(jnp.float32)
    g2 = gamma.astype(jnp.float32).reshape(Cout, 1)
    b2 = beta.astype(jnp.float32).reshape(Cout, 1)

    x_spec = pl.BlockSpec(
        (1, Cin, HW), lambda p, n: (jnp.where(p == 0, n, N - 1), 0, 0))
    o_spec = pl.BlockSpec(
        (1, Cout, HW), lambda p, n: (jnp.where(p == 0, 0, n), 0, 0))
    w_spec = pl.BlockSpec((Cout, Cin), lambda p, n: (0, 0))
    vec_spec = pl.BlockSpec((Cout, 1), lambda p, n: (0, 0))

    cost = pl.CostEstimate(
        flops=4 * M * Cin * Cout + 7 * M * Cout,
        transcendentals=Cout,
        bytes_accessed=M * Cin * 2 + M * Cout * 2 + Cout * Cin * 4)

    out_b = pl.pallas_call(
        _make_body(N, M, _BN_EPS),
        out_shape=jax.ShapeDtypeStruct((N, Cout, HW), jnp.bfloat16),
        grid=(2, N),
        in_specs=[x_spec, w_spec, vec_spec, vec_spec],
        out_specs=o_spec,
        scratch_shapes=[
            pltpu.VMEM((N, Cin, HW), jnp.bfloat16),
            pltpu.VMEM((Cin, Cin), jnp.float32),
            pltpu.VMEM((Cin, 128), jnp.float32),
            pltpu.VMEM((Cout, Cin), jnp.bfloat16),
            pltpu.VMEM((Cout, 1), jnp.float32),
        ],
        compiler_params=pltpu.CompilerParams(
            dimension_semantics=("arbitrary", "arbitrary"),
            vmem_limit_bytes=56 * 1024 * 1024),
        cost_estimate=cost,
    )(xb, w_mat, g2, b2)

    return out_b.astype(jnp.float32).reshape(N, Cout, H, W)


# speedup vs baseline: 6.2552x; 3.1408x over previous
import jax
import jax.numpy as jnp
from jax.experimental import pallas as pl
from jax.experimental.pallas import tpu as pltpu

_BN_EPS = 4e-5


def _make_body(hw, c_in, h, w, m_total, bn_eps):
    def _body(x_ref, w_ref, g_ref, b_ref, o_ref,
              x_buf, gram_ref, s_ref, ws_ref, shift_ref):
        p = pl.program_id(0)
        n = pl.program_id(1)

        @pl.when(p == 0)
        def _stats():
            @pl.when(n == 0)
            def _init():
                gram_ref[...] = jnp.zeros_like(gram_ref)
                s_ref[...] = jnp.zeros_like(s_ref)

            xm = x_ref[0].reshape(hw, c_in)
            x_buf[n] = xm.astype(jnp.bfloat16)
            gram_ref[...] += jax.lax.dot_general(
                xm, xm, (((0,), (0,)), ((), ())),
                preferred_element_type=jnp.float32)
            s_ref[...] += jnp.sum(xm, axis=0, keepdims=True)

        @pl.when(p == 1)
        def _normalize():
            @pl.when(n == 0)
            def _fold():
                wt = w_ref[...]
                mean = jnp.dot(s_ref[...], wt,
                               preferred_element_type=jnp.float32) / m_total
                a = jnp.dot(gram_ref[...], wt,
                            preferred_element_type=jnp.float32)
                e2 = jnp.sum(a * wt, axis=0, keepdims=True) / m_total
                var = jnp.maximum(e2 - mean * mean, 0.0)
                inv_std = 1.0 / jnp.sqrt(var + bn_eps)
                scale = g_ref[...] * inv_std
                shift_ref[...] = b_ref[...] - mean * scale
                ws_ref[...] = (wt * scale).astype(jnp.bfloat16)

            y = jnp.dot(x_buf[n], ws_ref[...],
                        preferred_element_type=jnp.float32)
            z = jnp.maximum(y + shift_ref[...], 0.0)
            o_ref[0] = z.astype(o_ref.dtype).reshape(h, w, -1)

    return _body


def kernel(x, conv_w, conv_b, gamma, beta):
    N, Cin, H, W = x.shape
    Cout = conv_w.shape[0]
    HW = H * W
    M = N * HW
    del conv_b

    xt = jnp.transpose(x, (0, 2, 3, 1))
    wt = conv_w.reshape(Cout, Cin).T.astype(jnp.float32)
    g2 = gamma.astype(jnp.float32).reshape(1, Cout)
    b2 = beta.astype(jnp.float32).reshape(1, Cout)

    x_spec = pl.BlockSpec(
        (1, H, W, Cin), lambda p, n: (jnp.where(p == 0, n, N - 1), 0, 0, 0))
    o_spec = pl.BlockSpec(
        (1, H, W, Cout), lambda p, n: (jnp.where(p == 0, 0, n), 0, 0, 0))
    w_spec = pl.BlockSpec((Cin, Cout), lambda p, n: (0, 0))
    vec_spec = pl.BlockSpec((1, Cout), lambda p, n: (0, 0))

    cost = pl.CostEstimate(
        flops=4 * M * Cin * Cout + 7 * M * Cout,
        transcendentals=Cout,
        bytes_accessed=M * Cin * 4 + M * Cout * 4 + Cout * Cin * 4)

    out = pl.pallas_call(
        _make_body(HW, Cin, H, W, M, _BN_EPS),
        out_shape=jax.ShapeDtypeStruct((N, H, W, Cout), x.dtype),
        grid=(2, N),
        in_specs=[x_spec, w_spec, vec_spec, vec_spec],
        out_specs=o_spec,
        scratch_shapes=[
            pltpu.VMEM((N, HW, Cin), jnp.bfloat16),
            pltpu.VMEM((Cin, Cin), jnp.float32),
            pltpu.VMEM((1, Cin), jnp.float32),
            pltpu.VMEM((Cin, Cout), jnp.bfloat16),
            pltpu.VMEM((1, Cout), jnp.float32),
        ],
        compiler_params=pltpu.CompilerParams(
            dimension_semantics=("arbitrary", "arbitrary"),
            vmem_limit_bytes=56 * 1024 * 1024),
        cost_estimate=cost,
    )(xt, wt, g2, b2)

    return jnp.transpose(out, (0, 3, 1, 2))
